# Initial kernel scaffold; baseline (speedup 1.0000x reference)
import jax
import jax.numpy as jnp
from jax.experimental import pallas as pl


def kernel(hidden_states, routing_weights, selected_experts, gate_proj, up_proj, down_proj):
    raise NotImplementedError("write your pallas kernel here")


# speedup vs baseline: 1.4100x; 1.4100x over previous
import jax
import jax.numpy as jnp
from jax.experimental import pallas as pl
from jax.experimental.pallas import tpu as pltpu

NUM_EXPERTS = 8
HIDDEN = 1024
INTER = 512
TOKENS = 2048
TOP_K = 2

T_TILE = 1024


def _moe_dense_body(x_ref, rw_ref, sel_ref, gate_ref, up_ref, down_ref, out_ref):
    j = pl.program_id(1)

    x = x_ref[...]
    g = jax.lax.dot_general(x, gate_ref[0], (((1,), (1,)), ((), ())),
                            preferred_element_type=jnp.float32)
    u = jax.lax.dot_general(x, up_ref[0], (((1,), (1,)), ((), ())),
                            preferred_element_type=jnp.float32)
    h = (g * jax.nn.sigmoid(g)) * u
    y = jax.lax.dot_general(h.astype(jnp.bfloat16), down_ref[0],
                            (((1,), (1,)), ((), ())),
                            preferred_element_type=jnp.float32)

    sel = sel_ref[...]
    rw = rw_ref[...]
    wcol = jnp.sum(jnp.where(sel == j, rw, 0.0), axis=1, keepdims=True)

    @pl.when(j == 0)
    def _():
        out_ref[...] = wcol * y

    @pl.when(j > 0)
    def _():
        out_ref[...] += wcol * y


def kernel(hidden_states, routing_weights, selected_experts, gate_proj, up_proj, down_proj):
    x16 = hidden_states.astype(jnp.bfloat16)
    g16 = gate_proj.astype(jnp.bfloat16)
    u16 = up_proj.astype(jnp.bfloat16)
    d16 = down_proj.astype(jnp.bfloat16)
    sel = selected_experts.astype(jnp.int32)

    n_t = TOKENS // T_TILE
    out = pl.pallas_call(
        _moe_dense_body,
        grid=(n_t, NUM_EXPERTS),
        in_specs=[
            pl.BlockSpec((T_TILE, HIDDEN), lambda i, j: (i, 0)),
            pl.BlockSpec((T_TILE, TOP_K), lambda i, j: (i, 0)),
            pl.BlockSpec((T_TILE, TOP_K), lambda i, j: (i, 0)),
            pl.BlockSpec((1, INTER, HIDDEN), lambda i, j: (j, 0, 0)),
            pl.BlockSpec((1, INTER, HIDDEN), lambda i, j: (j, 0, 0)),
            pl.BlockSpec((1, HIDDEN, INTER), lambda i, j: (j, 0, 0)),
        ],
        out_specs=pl.BlockSpec((T_TILE, HIDDEN), lambda i, j: (i, 0)),
        out_shape=jax.ShapeDtypeStruct((TOKENS, HIDDEN), jnp.float32),
        compiler_params=pltpu.CompilerParams(
            dimension_semantics=("parallel", "arbitrary"),
        ),
    )(x16, routing_weights, sel, g16, u16, d16)
    return out
